# Initial kernel scaffold; baseline (speedup 1.0000x reference)
import jax
import jax.numpy as jnp
from jax.experimental import pallas as pl


def kernel(attn_rpe_index, relative_position_bias_table):
    raise NotImplementedError("write your pallas kernel here")


# speedup vs baseline: 22.5527x; 22.5527x over previous
import jax
import jax.numpy as jnp
from jax import lax
from jax.experimental import pallas as pl
from jax.experimental.pallas import tpu as pltpu
from jax.experimental.pallas import tpu_sc as plsc

NUM_HEADS = 16
EMBED = 3969
ROWS = 1024
COLS = 1024
LANES = 16
NUM_WORKERS = 32
ROWS_PER_TILE = ROWS // NUM_WORKERS


def _gather_body(table_hbm, idx_hbm, out_hbm, table_v, idx_v0, idx_v1,
                 out_v0, out_v1, sem_tab, sem_in0, sem_in1, sem_out0,
                 sem_out1):
    idx_bufs = (idx_v0, idx_v1)
    out_bufs = (out_v0, out_v1)
    sems_in = (sem_in0, sem_in1)
    sems_out = (sem_out0, sem_out1)
    wid = lax.axis_index("s") * 2 + lax.axis_index("c")
    base = wid * ROWS_PER_TILE

    pltpu.async_copy(table_hbm, table_v, sem_tab).wait()

    pltpu.make_async_copy(
        idx_hbm.at[pl.ds(base, 1)], idx_bufs[0], sems_in[0]
    ).start()

    def row_pair(r2, carry):
        for b in range(2):
            r = r2 * 2 + b

            @pl.when(r + 1 < ROWS_PER_TILE)
            def _prefetch():
                pltpu.make_async_copy(
                    idx_hbm.at[pl.ds(base + r + 1, 1)],
                    idx_bufs[1 - b],
                    sems_in[1 - b],
                ).start()

            pltpu.make_async_copy(
                idx_hbm.at[pl.ds(base + r, 1)], idx_bufs[b], sems_in[b]
            ).wait()

            @pl.when(r2 >= 1)
            def _drain():
                pltpu.make_async_copy(
                    out_bufs[b],
                    out_hbm.at[:, pl.ds(base + r, 1), :],
                    sems_out[b],
                ).wait()

            def col_step(k, c):
                col = pl.multiple_of(k * LANES, LANES)
                iv = idx_bufs[b][0, pl.ds(col, LANES)]
                for h in range(NUM_HEADS):
                    vals = plsc.load_gather(table_v, [iv + h * EMBED])
                    out_bufs[b][h, 0, pl.ds(col, LANES)] = vals
                return c

            lax.fori_loop(0, COLS // LANES, col_step, 0)

            pltpu.make_async_copy(
                out_bufs[b],
                out_hbm.at[:, pl.ds(base + r, 1), :],
                sems_out[b],
            ).start()
        return carry

    lax.fori_loop(0, ROWS_PER_TILE // 2, row_pair, 0)

    for b in range(2):
        pltpu.make_async_copy(
            out_bufs[b],
            out_hbm.at[:, pl.ds(base + b, 1), :],
            sems_out[b],
        ).wait()


@jax.jit
def _rpe_gather(idx, table_flat):
    mesh = plsc.VectorSubcoreMesh(core_axis_name="c", subcore_axis_name="s")
    run = pl.kernel(
        _gather_body,
        out_type=jax.ShapeDtypeStruct((NUM_HEADS, ROWS, COLS), jnp.float32),
        mesh=mesh,
        compiler_params=pltpu.CompilerParams(needs_layout_passes=False),
        scratch_types=[
            pltpu.VMEM((NUM_HEADS * EMBED,), jnp.float32),
            pltpu.VMEM((1, COLS), jnp.int32),
            pltpu.VMEM((1, COLS), jnp.int32),
            pltpu.VMEM((NUM_HEADS, 1, COLS), jnp.float32),
            pltpu.VMEM((NUM_HEADS, 1, COLS), jnp.float32),
            pltpu.SemaphoreType.DMA,
            pltpu.SemaphoreType.DMA,
            pltpu.SemaphoreType.DMA,
            pltpu.SemaphoreType.DMA,
            pltpu.SemaphoreType.DMA,
        ],
    )
    return run(table_flat, idx)


def kernel(attn_rpe_index, relative_position_bias_table):
    idx = attn_rpe_index.astype(jnp.int32)
    table_flat = relative_position_bias_table.reshape(-1)
    out = _rpe_gather(idx, table_flat)
    return out[None]


# speedup vs baseline: 66.7439x; 2.9595x over previous
import jax
import jax.numpy as jnp
from jax import lax
from jax.experimental import pallas as pl
from jax.experimental.pallas import tpu as pltpu
from jax.experimental.pallas import tpu_sc as plsc

NUM_HEADS = 16
EMBED = 3969
ROWS = 1024
COLS = 1024
LANES = 16
NUM_WORKERS = 32
ROWS_PER_TILE = ROWS // NUM_WORKERS


def _gather_body(table_hbm, idx_hbm, out_hbm, table_v, idx_v0, idx_v1,
                 out_v0, out_v1, sem_tab, sem_in0, sem_in1, sem_out0,
                 sem_out1):
    idx_bufs = (idx_v0, idx_v1)
    out_bufs = (out_v0, out_v1)
    sems_in = (sem_in0, sem_in1)
    sems_out = (sem_out0, sem_out1)
    wid = lax.axis_index("s") * 2 + lax.axis_index("c")
    base = wid * ROWS_PER_TILE

    pltpu.async_copy(table_hbm, table_v, sem_tab).wait()

    pltpu.make_async_copy(
        idx_hbm.at[pl.ds(base, 1)], idx_bufs[0], sems_in[0]
    ).start()

    def row_pair(r2, carry):
        for b in range(2):
            r = r2 * 2 + b

            @pl.when(r + 1 < ROWS_PER_TILE)
            def _prefetch():
                pltpu.make_async_copy(
                    idx_hbm.at[pl.ds(base + r + 1, 1)],
                    idx_bufs[1 - b],
                    sems_in[1 - b],
                ).start()

            pltpu.make_async_copy(
                idx_hbm.at[pl.ds(base + r, 1)], idx_bufs[b], sems_in[b]
            ).wait()

            @pl.when(r2 >= 1)
            def _drain():
                pltpu.make_async_copy(
                    out_bufs[b],
                    out_hbm.at[:, pl.ds(base + r, 1), :],
                    sems_out[b],
                ).wait()

            @plsc.parallel_loop(0, COLS, step=LANES, unroll=4)
            def _cols(col):
                iv = idx_bufs[b][0, pl.ds(col, LANES)]
                for h in range(NUM_HEADS):
                    vals = plsc.load_gather(table_v, [iv + h * EMBED])
                    out_bufs[b][h, 0, pl.ds(col, LANES)] = vals

            pltpu.make_async_copy(
                out_bufs[b],
                out_hbm.at[:, pl.ds(base + r, 1), :],
                sems_out[b],
            ).start()
        return carry

    lax.fori_loop(0, ROWS_PER_TILE // 2, row_pair, 0)

    for b in range(2):
        pltpu.make_async_copy(
            out_bufs[b],
            out_hbm.at[:, pl.ds(base + b, 1), :],
            sems_out[b],
        ).wait()


@jax.jit
def _rpe_gather(idx, table_flat):
    mesh = plsc.VectorSubcoreMesh(core_axis_name="c", subcore_axis_name="s")
    run = pl.kernel(
        _gather_body,
        out_type=jax.ShapeDtypeStruct((NUM_HEADS, ROWS, COLS), jnp.float32),
        mesh=mesh,
        compiler_params=pltpu.CompilerParams(needs_layout_passes=False),
        scratch_types=[
            pltpu.VMEM((NUM_HEADS * EMBED,), jnp.float32),
            pltpu.VMEM((1, COLS), jnp.int32),
            pltpu.VMEM((1, COLS), jnp.int32),
            pltpu.VMEM((NUM_HEADS, 1, COLS), jnp.float32),
            pltpu.VMEM((NUM_HEADS, 1, COLS), jnp.float32),
            pltpu.SemaphoreType.DMA,
            pltpu.SemaphoreType.DMA,
            pltpu.SemaphoreType.DMA,
            pltpu.SemaphoreType.DMA,
            pltpu.SemaphoreType.DMA,
        ],
    )
    return run(table_flat, idx)


def kernel(attn_rpe_index, relative_position_bias_table):
    idx = attn_rpe_index.astype(jnp.int32)
    table_flat = relative_position_bias_table.reshape(-1)
    out = _rpe_gather(idx, table_flat)
    return out[None]
